# baseline (speedup 1.0000x reference)
import functools

import jax
import jax.numpy as jnp
from jax import lax
from jax.experimental import pallas as pl
from jax.experimental.pallas import tpu as pltpu
from jax.experimental.pallas import tpu_sc as plsc

_SC_BATCHES = 8
_SC_CHUNK = 16384


def _sc_sqrt(a):
    seed = jnp.where(
        a > 1e-2,
        jnp.float32(0.3),
        jnp.where(
            a > 1e-4,
            jnp.float32(0.03),
            jnp.where(a > 1e-6, jnp.float32(3e-3), jnp.float32(2e-4)),
        ),
    )
    x = jnp.broadcast_to(seed, a.shape)
    for _ in range(8):
        x = 0.5 * (x + a / x)
    return x


def _sc_dense_body(
    t_hbm, ab_hbm, x0_hbm, ns_hbm, out_hbm,
    t16, ab16, x0b, nsb, outb, semg, semx0, semx1, semn0, semn1,
    *, K, C, W, NC,
):
    cid = lax.axis_index("c")
    sid = lax.axis_index("s")
    wid = sid * NC + cid
    total = out_hbm.shape[0]
    elems = total // W
    spb = W // K
    bid = wid // spb
    base = wid * elems

    pltpu.async_copy(t_hbm.at[jnp.full((16,), bid, jnp.int32)], t16, semg).wait()
    pltpu.async_copy(ab_hbm.at[t16], ab16, semg).wait()
    ab = ab16[...]
    sa = _sc_sqrt(ab)
    sb = _sc_sqrt(1.0 - ab)

    sems = ((semx0, semn0), (semx1, semn1))

    def start(c, b):
        sx, sn = sems[b]
        h1 = pltpu.async_copy(x0_hbm.at[pl.ds(base + c * C, C)], x0b.at[b], sx)
        h2 = pltpu.async_copy(ns_hbm.at[pl.ds(base + c * C, C)], nsb.at[b], sn)
        return h1, h2

    nch = elems // C
    U = 8

    hs = start(0, 0)
    for c in range(nch):
        b = c & 1
        nxt = start(c + 1, 1 - b) if c + 1 < nch else None
        hs[0].wait()
        hs[1].wait()

        def body(i, _, b=b):
            for k in range(U):
                off = (i * U + k) * 16
                x = x0b[b, pl.ds(off, 16)]
                n = nsb[b, pl.ds(off, 16)]
                outb[b, pl.ds(off, 16)] = sa * x + sb * n
            return 0

        lax.fori_loop(0, C // (16 * U), body, 0)
        pltpu.sync_copy(outb.at[b], out_hbm.at[pl.ds(base + c * C, C)])
        hs = nxt


def _sc_dense(t, alpha_bars, x0f, nsf, K, C, n_per_batch):
    info = plsc.get_sparse_core_info()
    NC, NS = info.num_cores, info.num_subcores
    W = NC * NS
    mesh = plsc.VectorSubcoreMesh(core_axis_name="c", subcore_axis_name="s")
    n_out = K * n_per_batch
    fn = functools.partial(
        pl.kernel,
        mesh=mesh,
        out_type=jax.ShapeDtypeStruct((n_out,), jnp.float32),
        scratch_types=[
            pltpu.VMEM((16,), jnp.int32),
            pltpu.VMEM((16,), jnp.float32),
            pltpu.VMEM((2, C), jnp.float32),
            pltpu.VMEM((2, C), jnp.float32),
            pltpu.VMEM((2, C), jnp.float32),
            pltpu.SemaphoreType.DMA,
            pltpu.SemaphoreType.DMA,
            pltpu.SemaphoreType.DMA,
            pltpu.SemaphoreType.DMA,
            pltpu.SemaphoreType.DMA,
        ],
    )(functools.partial(_sc_dense_body, K=K, C=C, W=W, NC=NC))
    return fn(t, alpha_bars, x0f, nsf)


def _qsample_body(t_ref, ab_ref, x0_ref, noise_ref, out_ref, *, nb, k0):
    g = pl.program_id(0)
    for j in range(nb):
        b = k0 + g * nb + j
        ab = ab_ref[t_ref[b]]
        sa = jnp.sqrt(ab)
        sb = jnp.sqrt(1.0 - ab)
        out_ref[j] = sa * x0_ref[j] + sb * noise_ref[j]


@jax.jit
def kernel(x0, t, noise, alpha_bars):
    B, L, D = x0.shape
    K = _SC_BATCHES
    NB = 4

    x0f = x0.reshape(-1)
    nsf = noise.reshape(-1)
    sc_out = _sc_dense(t, alpha_bars, x0f, nsf, K, _SC_CHUNK, L * D)

    grid = ((B - K) // NB,)
    blk = pl.BlockSpec((NB, L, D), lambda g: (g + K // NB, 0, 0))
    tc_out = pl.pallas_call(
        functools.partial(_qsample_body, nb=NB, k0=K),
        grid=grid,
        in_specs=[
            pl.BlockSpec(memory_space=pltpu.SMEM),
            pl.BlockSpec(memory_space=pltpu.SMEM),
            blk,
            blk,
        ],
        out_specs=blk,
        out_shape=jax.ShapeDtypeStruct((B, L, D), jnp.float32),
        compiler_params=pltpu.CompilerParams(
            dimension_semantics=("parallel",),
        ),
    )(t, alpha_bars, x0, noise)

    return lax.dynamic_update_slice(tc_out, sc_out.reshape(K, L, D), (0, 0, 0))


# speedup vs baseline: 1.4015x; 1.4015x over previous
import functools

import jax
import jax.numpy as jnp
from jax.experimental import pallas as pl
from jax.experimental.pallas import tpu as pltpu


def _qsample_body(t_ref, ab_ref, x0_ref, noise_ref, out_ref, *, nb):
    g = pl.program_id(0)
    for j in range(nb):
        b = g * nb + j
        ab = ab_ref[t_ref[b]]
        sa = jnp.sqrt(ab)
        sb = jnp.sqrt(1.0 - ab)
        out_ref[j] = sa * x0_ref[j] + sb * noise_ref[j]


@jax.jit
def kernel(x0, t, noise, alpha_bars):
    B, L, D = x0.shape
    NB = 4
    grid = (B // NB,)
    blk = pl.BlockSpec((NB, L, D), lambda g: (g, 0, 0))
    return pl.pallas_call(
        functools.partial(_qsample_body, nb=NB),
        grid=grid,
        in_specs=[
            pl.BlockSpec(memory_space=pltpu.SMEM),
            pl.BlockSpec(memory_space=pltpu.SMEM),
            blk,
            blk,
        ],
        out_specs=blk,
        out_shape=jax.ShapeDtypeStruct((B, L, D), jnp.float32),
        compiler_params=pltpu.CompilerParams(
            dimension_semantics=("parallel",),
        ),
    )(t, alpha_bars, x0, noise)
